# baseline (speedup 1.0000x reference)
import functools

import jax
import jax.numpy as jnp
from jax import lax
from jax.experimental import pallas as pl
from jax.experimental.pallas import tpu as pltpu
from jax.experimental.pallas import tpu_sc as plsc

EMBED = 64
HIDDEN = 256
OUT = 2
N_TOKENS = 16384

NC = 2
NS = 16
NW = NC * NS
PER_TILE = N_TOKENS // NW
CHUNK = 128
NCHUNK = PER_TILE // CHUNK
LANES = 16
VECS = EMBED // LANES


def _sc_gather_sum(table, idx):
    mesh = plsc.VectorSubcoreMesh(core_axis_name="c", subcore_axis_name="s")

    @functools.partial(
        pl.kernel,
        mesh=mesh,
        out_type=jax.ShapeDtypeStruct((NC, EMBED), jnp.float32),
        scratch_types=[
            pltpu.VMEM((NCHUNK, CHUNK), jnp.int32),
            pltpu.VMEM((PER_TILE, EMBED), jnp.float32),
            pltpu.VMEM((EMBED,), jnp.float32),
            pltpu.VMEM_SHARED((NS, EMBED), jnp.float32),
            pltpu.VMEM((NS, EMBED), jnp.float32),
            pltpu.SemaphoreType.DMA,
        ],
        compiler_params=pltpu.CompilerParams(use_tc_tiling_on_sc=False),
    )
    def k(table_hbm, idx_hbm, out_hbm, idx_v, rows_v, part_v, shared, gath_v, sem):
        cid = lax.axis_index("c")
        sid = lax.axis_index("s")
        wid = sid * NC + cid

        pltpu.sync_copy(idx_hbm.at[wid], idx_v)

        copies = []
        for g in range(NCHUNK):
            copies.append(
                pltpu.async_copy(
                    table_hbm.at[idx_v.at[g]],
                    rows_v.at[pl.ds(g * CHUNK, CHUNK)],
                    sem,
                )
            )
        for c in copies:
            c.wait()

        def body(r, accs):
            return tuple(
                accs[j] + rows_v[r, pl.ds(j * LANES, LANES)] for j in range(VECS)
            )

        zeros = tuple(jnp.zeros((LANES,), jnp.float32) for _ in range(VECS))
        accs = lax.fori_loop(0, PER_TILE, body, zeros)
        for j in range(VECS):
            part_v[pl.ds(j * LANES, LANES)] = accs[j]

        pltpu.sync_copy(part_v, shared.at[sid])
        plsc.subcore_barrier()

        @pl.when(sid == 0)
        def _():
            pltpu.sync_copy(shared, gath_v)

            def body2(r, accs):
                return tuple(
                    accs[j] + gath_v[r, pl.ds(j * LANES, LANES)] for j in range(VECS)
                )

            accs2 = lax.fori_loop(0, NS, body2, zeros)
            for j in range(VECS):
                part_v[pl.ds(j * LANES, LANES)] = accs2[j]
            pltpu.sync_copy(part_v, out_hbm.at[cid])

    return k(table, idx)


def _tc_mlp(partials, W1, b1col, W2T, b2row):
    def mlp_kernel(p_ref, w1_ref, b1_ref, w2t_ref, b2_ref, o_ref):
        avg = jnp.sum(p_ref[...], axis=0, keepdims=True) * (1.0 / N_TOKENS)
        t1 = w1_ref[...] * avg
        h = jnp.maximum(jnp.sum(t1, axis=1, keepdims=True) + b1_ref[...], 0.0)
        t2 = w2t_ref[...] * h
        o_ref[...] = jnp.sum(t2, axis=0, keepdims=True) + b2_ref[...]

    return pl.pallas_call(
        mlp_kernel,
        out_shape=jax.ShapeDtypeStruct((1, OUT), jnp.float32),
    )(partials, W1, b1col, W2T, b2row)


def kernel(indices, table, W1, b1, W2, b2):
    idx = indices.astype(jnp.int32).reshape(NW, NCHUNK, CHUNK)
    partials = _sc_gather_sum(table, idx)
    logits = _tc_mlp(partials, W1, b1[:, None], W2.T, b2[None, :])
    return logits.reshape(OUT)


# speedup vs baseline: 1.6783x; 1.6783x over previous
import functools

import jax
import jax.numpy as jnp
from jax import lax
from jax.experimental import pallas as pl
from jax.experimental.pallas import tpu as pltpu
from jax.experimental.pallas import tpu_sc as plsc

EMBED = 64
HIDDEN = 256
OUT = 2
N_TOKENS = 16384

NC = 2
NS = 16
NW = NC * NS
PER_TILE = N_TOKENS // NW
CS = 16
NCHU = PER_TILE // CS
LANES = 16
VECS = EMBED // LANES


def _sc_gather_sum(table, idx):
    mesh = plsc.VectorSubcoreMesh(core_axis_name="c", subcore_axis_name="s")

    @functools.partial(
        pl.kernel,
        mesh=mesh,
        out_type=jax.ShapeDtypeStruct((NW, EMBED), jnp.float32),
        scratch_types=[
            pltpu.VMEM((PER_TILE,), jnp.int32),
            pltpu.VMEM((CS, EMBED), jnp.float32),
            pltpu.VMEM((CS, EMBED), jnp.float32),
            pltpu.VMEM((EMBED,), jnp.float32),
            pltpu.SemaphoreType.DMA,
            pltpu.SemaphoreType.DMA,
        ],
    )
    def k(table_hbm, idx_hbm, out_hbm,
          idx_v, bufA, bufB, part_v, semA, semB):
        cid = lax.axis_index("c")
        sid = lax.axis_index("s")
        wid = sid * NC + cid

        pltpu.sync_copy(idx_hbm.at[wid], idx_v)

        bufs = (bufA, bufB)
        sems = (semA, semB)

        def fire(c):
            buf, sem = bufs[c % 2], sems[c % 2]
            iv = idx_v[pl.ds(c * CS, LANES)]
            return [
                pltpu.async_copy(table_hbm.at[iv[r]], buf.at[r], sem)
                for r in range(CS)
            ]

        def drain(handles):
            for h in handles:
                h.wait()

        def reduce_chunk(c, accs):
            buf = bufs[c % 2]

            def body(r, accs):
                return tuple(
                    accs[j] + buf[r, pl.ds(j * LANES, LANES)]
                    for j in range(VECS)
                )

            return lax.fori_loop(0, CS, body, accs)

        accs = tuple(jnp.zeros((LANES,), jnp.float32) for _ in range(VECS))
        hs = fire(0)
        for c in range(NCHU):
            nxt = fire(c + 1) if c + 1 < NCHU else None
            drain(hs)
            accs = reduce_chunk(c, accs)
            hs = nxt

        for j in range(VECS):
            part_v[pl.ds(j * LANES, LANES)] = accs[j]

        pltpu.sync_copy(part_v, out_hbm.at[wid])

    return k(table, idx)


def _tc_mlp(partials, W1, b1col, W2T, b2row):
    def mlp_kernel(p_ref, w1_ref, b1_ref, w2t_ref, b2_ref, o_ref):
        avg = jnp.sum(p_ref[...], axis=0, keepdims=True) * (1.0 / N_TOKENS)
        t1 = w1_ref[...] * avg
        h = jnp.maximum(jnp.sum(t1, axis=1, keepdims=True) + b1_ref[...], 0.0)
        t2 = w2t_ref[...] * h
        o_ref[...] = jnp.sum(t2, axis=0, keepdims=True) + b2_ref[...]

    return pl.pallas_call(
        mlp_kernel,
        out_shape=jax.ShapeDtypeStruct((1, OUT), jnp.float32),
    )(partials, W1, b1col, W2T, b2row)


def kernel(indices, table, W1, b1, W2, b2):
    idx = indices.astype(jnp.int32).reshape(NW, PER_TILE)
    partials = _sc_gather_sum(table, idx)
    logits = _tc_mlp(partials, W1, b1[:, None], W2.T, b2[None, :])
    return logits.reshape(OUT)
